# Initial kernel scaffold; baseline (speedup 1.0000x reference)
import jax
import jax.numpy as jnp
from jax.experimental import pallas as pl


def kernel(node_feat, edge_feat, edge_index, n_node, n_edge, enc_node, enc_edge, proc, dec):
    raise NotImplementedError("write your pallas kernel here")


# speedup vs baseline: 6.6561x; 6.6561x over previous
import functools

import jax
import jax.numpy as jnp
from jax import lax
from jax.experimental import pallas as pl
from jax.experimental.pallas import tpu as pltpu
from jax.experimental.pallas import tpu_sc as plsc

_NC = 2
_NS = 16
_NW = _NC * _NS
_EPS = 1e-5
_HP = jax.lax.Precision.HIGHEST


def _mesh():
    return plsc.VectorSubcoreMesh(core_axis_name="c", subcore_axis_name="s")


def _collapse3(p1, p2, p3):
    (w1, b1), (w2, b2), (w3, b3) = p1, p2, p3
    K = w1.shape[0]
    O = w3.shape[1]

    hp = jax.lax.Precision.HIGHEST

    def body(w1_r, b1_r, w2_r, b2_r, w3_r, b3_r, wo_r, bo_r):
        w12 = jnp.dot(w1_r[...], w2_r[...], precision=hp,
                      preferred_element_type=jnp.float32)
        wo_r[...] = jnp.dot(w12, w3_r[...], precision=hp,
                            preferred_element_type=jnp.float32)
        b12 = jnp.dot(b1_r[...], w2_r[...], precision=hp,
                      preferred_element_type=jnp.float32) + b2_r[...]
        bo_r[...] = jnp.dot(b12, w3_r[...], precision=hp,
                            preferred_element_type=jnp.float32) + b3_r[...]

    return pl.pallas_call(
        body,
        out_shape=(jax.ShapeDtypeStruct((K, O), jnp.float32),
                   jax.ShapeDtypeStruct((1, O), jnp.float32)),
    )(w1, b1.reshape(1, -1), w2, b2.reshape(1, -1), w3, b3.reshape(1, -1))


def _ln_block(t, g, b):
    mu = jnp.mean(t, axis=-1, keepdims=True)
    var = jnp.mean((t - mu) ** 2, axis=-1, keepdims=True)
    return (t - mu) / jnp.sqrt(var + _EPS) * g + b


def _linear_ln(x, w, b2d, g2d, be2d, block_rows):
    R, K = x.shape
    O = w.shape[1]

    def body(x_r, w_r, b_r, g_r, be_r, o_r):
        t = jnp.dot(x_r[...], w_r[...], precision=_HP, preferred_element_type=jnp.float32) + b_r[...]
        o_r[...] = _ln_block(t, g_r[...], be_r[...])

    return pl.pallas_call(
        body,
        grid=(R // block_rows,),
        in_specs=[
            pl.BlockSpec((block_rows, K), lambda i: (i, 0)),
            pl.BlockSpec((K, O), lambda i: (0, 0)),
            pl.BlockSpec((1, O), lambda i: (0, 0)),
            pl.BlockSpec((1, O), lambda i: (0, 0)),
            pl.BlockSpec((1, O), lambda i: (0, 0)),
        ],
        out_specs=pl.BlockSpec((block_rows, O), lambda i: (i, 0)),
        out_shape=jax.ShapeDtypeStruct((R, O), jnp.float32),
    )(x, w, b2d, g2d, be2d)


def _encode_combine(h_n0, sums, cnts, block_rows):
    Nn, D = h_n0.shape

    def body(h_r, s_r, c_r, o_r, inv_r):
        c = c_r[0, :, 0:1] + c_r[1, :, 0:1]
        inv = 1.0 / jnp.maximum(c, 1.0)
        o_r[...] = h_r[...] + (s_r[0] + s_r[1]) * inv
        inv_r[...] = jnp.broadcast_to(inv, h_r.shape)

    return pl.pallas_call(
        body,
        grid=(Nn // block_rows,),
        in_specs=[
            pl.BlockSpec((block_rows, D), lambda i: (i, 0)),
            pl.BlockSpec((2, block_rows, D), lambda i: (0, i, 0)),
            pl.BlockSpec((2, block_rows, 128), lambda i: (0, i, 0)),
        ],
        out_specs=(pl.BlockSpec((block_rows, D), lambda i: (i, 0)),
                   pl.BlockSpec((block_rows, D), lambda i: (i, 0))),
        out_shape=(jax.ShapeDtypeStruct((Nn, D), jnp.float32),
                   jax.ShapeDtypeStruct((Nn, D), jnp.float32)),
    )(h_n0, sums, cnts)


def _proj2(h_n, ws, wd, block_rows):
    Nn, D = h_n.shape

    def body(h_r, ws_r, wd_r, o1_r, o2_r):
        h = h_r[...]
        o1_r[...] = jnp.dot(h, ws_r[...], precision=_HP, preferred_element_type=jnp.float32)
        o2_r[...] = jnp.dot(h, wd_r[...], precision=_HP, preferred_element_type=jnp.float32)

    return pl.pallas_call(
        body,
        grid=(Nn // block_rows,),
        in_specs=[
            pl.BlockSpec((block_rows, D), lambda i: (i, 0)),
            pl.BlockSpec((D, D), lambda i: (0, 0)),
            pl.BlockSpec((D, D), lambda i: (0, 0)),
        ],
        out_specs=(pl.BlockSpec((block_rows, D), lambda i: (i, 0)),
                   pl.BlockSpec((block_rows, D), lambda i: (i, 0))),
        out_shape=(jax.ShapeDtypeStruct((Nn, D), jnp.float32),
                   jax.ShapeDtypeStruct((Nn, D), jnp.float32)),
    )(h_n, ws, wd)


def _edge_step(h_e, g1, g2, we, b2d, lng, lnb, block_rows):
    Ee, D = h_e.shape

    def body(h_r, g1_r, g2_r, w_r, b_r, gg_r, bb_r, ne_r, ho_r):
        h = h_r[...]
        t = jnp.dot(h, w_r[...], precision=_HP, preferred_element_type=jnp.float32)
        t = t + g1_r[...] + g2_r[...] + b_r[...]
        ne = _ln_block(t, gg_r[...], bb_r[...])
        ne_r[...] = ne
        ho_r[...] = h + ne

    return pl.pallas_call(
        body,
        grid=(Ee // block_rows,),
        in_specs=[
            pl.BlockSpec((block_rows, D), lambda i: (i, 0)),
            pl.BlockSpec((block_rows, D), lambda i: (i, 0)),
            pl.BlockSpec((block_rows, D), lambda i: (i, 0)),
            pl.BlockSpec((D, D), lambda i: (0, 0)),
            pl.BlockSpec((1, D), lambda i: (0, 0)),
            pl.BlockSpec((1, D), lambda i: (0, 0)),
            pl.BlockSpec((1, D), lambda i: (0, 0)),
        ],
        out_specs=(pl.BlockSpec((block_rows, D), lambda i: (i, 0)),
                   pl.BlockSpec((block_rows, D), lambda i: (i, 0))),
        out_shape=(jax.ShapeDtypeStruct((Ee, D), jnp.float32),
                   jax.ShapeDtypeStruct((Ee, D), jnp.float32)),
    )(h_e, g1, g2, we, b2d, lng, lnb)


def _node_step(h_n, sums, inv, vh, va, b2d, lng, lnb, block_rows):
    Nn, D = h_n.shape

    def body(h_r, s_r, i_r, vh_r, va_r, b_r, gg_r, bb_r, o_r):
        h = h_r[...]
        agg = (s_r[0] + s_r[1]) * i_r[...]
        t = (jnp.dot(h, vh_r[...], precision=_HP, preferred_element_type=jnp.float32)
             + jnp.dot(agg, va_r[...], precision=_HP, preferred_element_type=jnp.float32)
             + b_r[...])
        o_r[...] = h + _ln_block(t, gg_r[...], bb_r[...])

    return pl.pallas_call(
        body,
        grid=(Nn // block_rows,),
        in_specs=[
            pl.BlockSpec((block_rows, D), lambda i: (i, 0)),
            pl.BlockSpec((2, block_rows, D), lambda i: (0, i, 0)),
            pl.BlockSpec((block_rows, D), lambda i: (i, 0)),
            pl.BlockSpec((D, D), lambda i: (0, 0)),
            pl.BlockSpec((D, D), lambda i: (0, 0)),
            pl.BlockSpec((1, D), lambda i: (0, 0)),
            pl.BlockSpec((1, D), lambda i: (0, 0)),
            pl.BlockSpec((1, D), lambda i: (0, 0)),
        ],
        out_specs=pl.BlockSpec((block_rows, D), lambda i: (i, 0)),
        out_shape=jax.ShapeDtypeStruct((Nn, D), jnp.float32),
    )(h_n, sums, inv, vh, va, b2d, lng, lnb)


def _linear(x, w, b2d, block_rows):
    R, K = x.shape
    O = w.shape[1]

    def body(x_r, w_r, b_r, o_r):
        o_r[...] = jnp.dot(x_r[...], w_r[...], precision=_HP, preferred_element_type=jnp.float32) + b_r[...]

    return pl.pallas_call(
        body,
        grid=(R // block_rows,),
        in_specs=[
            pl.BlockSpec((block_rows, K), lambda i: (i, 0)),
            pl.BlockSpec((K, O), lambda i: (0, 0)),
            pl.BlockSpec((1, O), lambda i: (0, 0)),
        ],
        out_specs=pl.BlockSpec((block_rows, O), lambda i: (i, 0)),
        out_shape=jax.ShapeDtypeStruct((R, O), jnp.float32),
    )(x, w, b2d)


_GW = 80


def _sc_gather2(ps, pd, src3, dst3):
    nwin = src3.shape[0]
    Ee = nwin * _GW
    D = ps.shape[1]

    @functools.partial(
        pl.kernel,
        out_type=(jax.ShapeDtypeStruct((Ee, D), jnp.float32),
                  jax.ShapeDtypeStruct((Ee, D), jnp.float32)),
        mesh=_mesh(),
    )
    def k(ps_hbm, pd_hbm, src_hbm, dst_hbm, o1_hbm, o2_hbm):
        def body(src_v, dst_v, o1_v, o2_v):
            pltpu.sync_copy(ps_hbm.at[src_v.at[0, 0]], o1_v)
            pltpu.sync_copy(pd_hbm.at[dst_v.at[0, 0]], o2_v)

        pltpu.emit_pipeline(
            body,
            grid=(nwin,),
            in_specs=[pl.BlockSpec((1, 1, _GW), lambda i: (i, 0, 0)),
                      pl.BlockSpec((1, 1, _GW), lambda i: (i, 0, 0))],
            out_specs=[pl.BlockSpec((_GW, D), lambda i: (i, 0)),
                       pl.BlockSpec((_GW, D), lambda i: (i, 0))],
            core_axis_name=("c", "s"),
            dimension_semantics=(pltpu.PARALLEL,),
        )(src_hbm, dst_hbm, o1_hbm, o2_hbm)

    return k(ps, pd, src3, dst3)


def _sc_scatter_jnp_debug(rows, idx4, n_nodes, with_cnt):
    idx = idx4.reshape(-1)
    half = rows.shape[0] // 2
    s0 = jax.ops.segment_sum(rows[:half], idx[:half], num_segments=n_nodes)
    s1 = jax.ops.segment_sum(rows[half:], idx[half:], num_segments=n_nodes)
    sums = jnp.stack([s0, s1])
    if not with_cnt:
        return sums
    ones = jnp.ones((rows.shape[0], 16), jnp.float32)
    c0 = jax.ops.segment_sum(ones[:half], idx[:half], num_segments=n_nodes)
    c1 = jax.ops.segment_sum(ones[half:], idx[half:], num_segments=n_nodes)
    return sums, jnp.stack([c0, c1])


def _sc_scatter(rows, idx5, n_nodes, D=None):
    if rows is not None:
        Ee, D = rows.shape
    nwk, nsec, sec, C = idx5.shape
    per_w = nsec * sec * C
    nnch = n_nodes // C
    kmax = (nnch + _NS - 1) // _NS

    scratch = [
        pltpu.VMEM((C, D), jnp.float32),
        pltpu.VMEM((sec, C), jnp.int32),
        pltpu.VMEM_SHARED((n_nodes, D), jnp.float32),
    ]

    def body(*args):
        if rows is not None:
            rows_hbm, idx_hbm, sums_hbm, rows_v, idx_v, acc_sh = args
        else:
            idx_hbm, sums_hbm, rows_v, idx_v, acc_sh = args
        cid = lax.axis_index("c")
        sid = lax.axis_index("s")
        wid = cid * _NS + sid

        def fill(val):
            @pl.loop(0, C)
            def _fr(r):
                @pl.loop(0, D // 16)
                def _fc(c):
                    rows_v[r, pl.ds(c * 16, 16)] = jnp.full((16,), val,
                                                            jnp.float32)

        fill(0.0)

        @pl.loop(0, kmax)
        def _za(kk):
            ch = sid + kk * _NS

            @pl.when(ch < nnch)
            def _():
                pltpu.sync_copy(rows_v, acc_sh.at[pl.ds(ch * C, C)])

        if rows is None:
            fill(1.0)
        plsc.subcore_barrier()

        base = wid * per_w

        @pl.loop(0, nsec)
        def _sec(s):
            pltpu.sync_copy(idx_hbm.at[wid, s], idx_v)

            @pl.loop(0, sec)
            def _acc(j):
                if rows is not None:
                    pltpu.sync_copy(
                        rows_hbm.at[pl.ds(base + (s * sec + j) * C, C)], rows_v)
                pltpu.sync_copy(rows_v, acc_sh.at[idx_v.at[j]], add=True)

        plsc.subcore_barrier()

        @pl.loop(0, kmax)
        def _rd(kk):
            ch = sid + kk * _NS

            @pl.when(ch < nnch)
            def _():
                pltpu.sync_copy(acc_sh.at[pl.ds(ch * C, C)],
                                sums_hbm.at[cid, pl.ds(ch * C, C)])

    k = pl.kernel(body,
                  out_type=jax.ShapeDtypeStruct((_NC, n_nodes, D), jnp.float32),
                  mesh=_mesh(), scratch_types=scratch)
    return k(rows, idx5) if rows is not None else k(idx5)


def kernel(node_feat, edge_feat, edge_index, n_node, n_edge,
           enc_node, enc_edge, proc, dec):
    Nn, DN = node_feat.shape
    Ee, DE = edge_feat.shape
    D = enc_node["ln"][0].shape[0]

    BN = 2000
    BE = 3200

    src = edge_index[0]
    dst = edge_index[1]
    nch = Ee // (_NW * _GW)
    nsec = 5
    idx5 = src.reshape(_NW, nsec, nch // nsec, _GW)
    src3 = src.reshape(Ee // _GW, 1, _GW)
    dst3 = dst.reshape(Ee // _GW, 1, _GW)

    def r2(v):
        return v.reshape(1, -1)

    wn, bn = _collapse3(*enc_node["mlp"])
    we, be = _collapse3(*enc_edge["mlp"])
    steps = []
    for p in proc:
        wcat, bcat = _collapse3(*p["edge_mlp"])
        vcat, vb = _collapse3(*p["node_mlp"])
        steps.append(dict(
            w_e=wcat[:D], w_s=wcat[D:2 * D], w_d=wcat[2 * D:],
            b=bcat,
            e_g=r2(p["edge_ln"][0]), e_b=r2(p["edge_ln"][1]),
            vh=vcat[:D], va=vcat[D:],
            vb=vb,
            n_g=r2(p["node_ln"][0]), n_b=r2(p["node_ln"][1]),
        ))
    (dw1, db1), (dw2, db2), (dw3, db3) = dec
    dw3p = jnp.pad(dw3, ((0, 0), (0, D - dw3.shape[1])))
    db3p = jnp.pad(db3, (0, D - db3.shape[0]))
    wdec, bdec = _collapse3((dw1, db1), (dw2, db2), (dw3p, db3p))

    h_n0 = _linear_ln(node_feat, wn, bn, r2(enc_node["ln"][0]),
                      r2(enc_node["ln"][1]), BN)
    h_e = _linear_ln(edge_feat, we, be, r2(enc_edge["ln"][0]),
                     r2(enc_edge["ln"][1]), BE)
    sums = _sc_scatter(h_e, idx5, Nn)
    cnts = _sc_scatter(None, idx5, Nn, D=D)
    h_n, inv = _encode_combine(h_n0, sums, cnts, BN)

    for st in steps:
        ps, pdd = _proj2(h_n, st["w_s"], st["w_d"], BN)
        g1, g2 = _sc_gather2(ps, pdd, src3, dst3)
        new_e, h_e = _edge_step(h_e, g1, g2, st["w_e"], st["b"],
                                st["e_g"], st["e_b"], BE)
        s2 = _sc_scatter(new_e, idx5, Nn)
        h_n = _node_step(h_n, s2, inv, st["vh"], st["va"], st["vb"],
                         st["n_g"], st["n_b"], BN)

    out = _linear(h_n, wdec, bdec, BN)
    return out[:, :dw3.shape[1]]
